# baseline (speedup 1.0000x reference)
import functools

import jax
import jax.numpy as jnp
from jax import lax
from jax.experimental import pallas as pl
from jax.experimental.pallas import tpu as pltpu
from jax.experimental.pallas import tpu_sc as plsc

_LAMADA = 0.001
_B = 16384
_D = 64
_NC = 2
_NS = 16
_NW = _NC * _NS
_BPW = _B // _NW
_CHUNK = 128
_NCHUNK = _BPW // _CHUNK
_UNROLL = 16


def _tile_body(user0_hbm, item0_hbm, ratings_hbm, euser_hbm, eitem_hbm,
               out_hbm, idx_u, idx_i, urows, irows, rat, res, sem):
    wid = lax.axis_index("s") * _NC + lax.axis_index("c")
    base = wid * _BPW

    pltpu.sync_copy(user0_hbm.at[pl.ds(base, _BPW)], idx_u)
    pltpu.sync_copy(item0_hbm.at[pl.ds(base, _BPW)], idx_i)
    pltpu.sync_copy(ratings_hbm.at[pl.ds(base, _BPW)], rat)

    copies = []
    for j in range(_NCHUNK):
        sl = pl.ds(j * _CHUNK, _CHUNK)
        copies.append(pltpu.async_copy(euser_hbm.at[idx_u.at[sl]], urows.at[sl], sem))
        copies.append(pltpu.async_copy(eitem_hbm.at[idx_i.at[sl]], irows.at[sl], sem))
    for c in copies:
        c.wait()

    lane = lax.iota(jnp.int32, 16)
    zeros = jnp.zeros((16,), jnp.float32)

    def row_block(blk, carry):
        loss2_acc, u2_acc, i2_acc = carry
        rv = rat[pl.ds(blk * _UNROLL, 16)]
        for k in range(_UNROLL):
            r = blk * _UNROLL + k
            us = [urows[r, pl.ds(16 * c, 16)] for c in range(_D // 16)]
            vs = [irows[r, pl.ds(16 * c, 16)] for c in range(_D // 16)]
            t = us[0] * vs[0]
            for c in range(1, _D // 16):
                t = t + us[c] * vs[c]
            err = jnp.sum(t) - rv[k]
            loss2_acc = loss2_acc + err * err
            for c in range(_D // 16):
                u2_acc = u2_acc + us[c] * us[c]
                i2_acc = i2_acc + vs[c] * vs[c]
        return (loss2_acc, u2_acc, i2_acc)

    loss2_s, u2_acc, i2_acc = lax.fori_loop(
        0, _BPW // _UNROLL, row_block, (jnp.float32(0.0), zeros, zeros))

    loss2_s = loss2_s * (1.0 / _B)
    u2_s = jnp.sum(u2_acc) * (_LAMADA / (_B * _D))
    i2_s = jnp.sum(i2_acc) * (_LAMADA / (_B * _D))

    vec = (jnp.where(lane == 0, jnp.full((16,), loss2_s), zeros)
           + jnp.where(lane == 1, jnp.full((16,), u2_s), zeros)
           + jnp.where(lane == 2, jnp.full((16,), i2_s), zeros))
    res[...] = vec
    pltpu.sync_copy(res, out_hbm.at[wid])


@jax.jit
def _bpr_partials(user0, item_i0, ratings, embed_user, embed_item):
    mesh = plsc.VectorSubcoreMesh(core_axis_name="c", subcore_axis_name="s")
    kfn = functools.partial(
        pl.kernel,
        out_type=jax.ShapeDtypeStruct((_NW, 16), jnp.float32),
        mesh=mesh,
        compiler_params=pltpu.CompilerParams(
            needs_layout_passes=False, use_tc_tiling_on_sc=False),
        scratch_types=[
            pltpu.VMEM((_BPW,), jnp.int32),
            pltpu.VMEM((_BPW,), jnp.int32),
            pltpu.VMEM((_BPW, _D), jnp.float32),
            pltpu.VMEM((_BPW, _D), jnp.float32),
            pltpu.VMEM((_BPW,), jnp.float32),
            pltpu.VMEM((16,), jnp.float32),
            pltpu.SemaphoreType.DMA,
        ],
    )(_tile_body)
    return kfn(user0, item_i0, ratings, embed_user, embed_item)


def kernel(user0, item_i0, ratings, embed_user, embed_item):
    parts = _bpr_partials(user0, item_i0, ratings, embed_user, embed_item)
    s = jnp.sum(parts, axis=0)
    loss2 = s[0]
    l2 = s[1] + s[2]
    return (loss2 + l2, loss2, l2)
